# baseline (speedup 1.0000x reference)
import functools

import jax
import jax.numpy as jnp
from jax import lax
from jax.experimental import pallas as pl
from jax.experimental.pallas import tpu as pltpu
from jax.experimental.pallas import tpu_sc as plsc


def _linearize_body(x_ref, out_ref):
    xb = x_ref[0]
    pieces = [xb[None, :, 128 * k:128 * (k + 1)] for k in range(6)]
    out_ref[...] = jnp.concatenate(pieces, axis=0)


def _linearize_call(x, B, T, D):
    TB = 2048
    n_blk = (B * T) // TB
    x3 = x.reshape(n_blk, TB, D)
    return pl.pallas_call(
        _linearize_body,
        grid=(n_blk,),
        in_specs=[pl.BlockSpec((1, TB, D), lambda i: (i, 0, 0))],
        out_specs=pl.BlockSpec((6, TB, 128), lambda i: (0, i, 0)),
        out_shape=jax.ShapeDtypeStruct((6, B * T, 128), jnp.float32),
    )(x3)


def _partner(arr, bit, axis):
    fwd = jnp.roll(arr, -bit, axis=axis)
    bwd = jnp.roll(arr, bit, axis=axis)
    io = lax.broadcasted_iota(jnp.int32, arr.shape, axis)
    take_fwd = (io & bit) == 0
    return jnp.where(take_fwd, fwd, bwd)


def _bitonic(keys, idxs, n, rows, lanes, descending):
    shape = keys.shape
    row_io = lax.broadcasted_iota(jnp.int32, shape, 1)
    lane_io = lax.broadcasted_iota(jnp.int32, shape, 2)
    flat_io = row_io * lanes + lane_io

    k = 2
    while k <= n:
        j = k // 2
        while j >= 1:
            if j < lanes:
                kp = _partner(keys, j, 2)
                ip = _partner(idxs, j, 2)
            else:
                rj = j // lanes
                kp = _partner(keys, rj, 1)
                ip = _partner(idxs, rj, 1)
            own_lower = (flat_io & j) == 0
            up = (flat_io & k) == 0
            if descending:
                own_first = (keys > kp) | ((keys == kp) & (idxs < ip))
            else:
                own_first = (keys < kp) | ((keys == kp) & (idxs < ip))
            keep_own = own_first == (own_lower == up)
            keys = jnp.where(keep_own, keys, kp)
            idxs = jnp.where(keep_own, idxs, ip)
            j //= 2
        k *= 2
    return keys, idxs


def _sort_body(T, K_SKIP, logits_ref, perm_ref, skipg_ref, skipw_ref):
    R = T // 128
    v = logits_ref[...]
    row_io = lax.broadcasted_iota(jnp.int32, v.shape, 1)
    lane_io = lax.broadcasted_iota(jnp.int32, v.shape, 2)
    idx = row_io * 128 + lane_io

    vs, isrt = _bitonic(v, idx, T, R, 128, descending=True)
    perm_ref[...] = isrt

    tv = vs[:, R - 8:, :]
    ti = isrt[:, R - 8:, :]
    tvs, tis = _bitonic(tv, ti, 1024, 8, 128, descending=False)
    skipg_ref[...] = tis

    fr = lax.broadcasted_iota(jnp.int32, tvs.shape, 1)
    fc = lax.broadcasted_iota(jnp.int32, tvs.shape, 2)
    fflat = fr * 128 + fc
    mask = fflat < K_SKIP
    mrow = (K_SKIP - 1) // 128
    mcol = (K_SKIP - 1) % 128
    m = tvs[:, mrow:mrow + 1, mcol:mcol + 1]
    e = jnp.exp(jnp.where(mask, tvs - m, -jnp.inf))
    s = jnp.sum(e, axis=(1, 2), keepdims=True)
    skipw_ref[...] = e / s


def _sort_call(logits3, B, T, K_SKIP):
    body = functools.partial(_sort_body, T, K_SKIP)
    R = T // 128
    perm, skip_gid, skip_w = pl.pallas_call(
        body,
        grid=(B,),
        in_specs=[pl.BlockSpec((1, R, 128), lambda b: (b, 0, 0))],
        out_specs=[
            pl.BlockSpec((1, R, 128), lambda b: (b, 0, 0)),
            pl.BlockSpec((1, 8, 128), lambda b: (b, 0, 0)),
            pl.BlockSpec((1, 8, 128), lambda b: (b, 0, 0)),
        ],
        out_shape=[
            jax.ShapeDtypeStruct((B, R, 128), jnp.int32),
            jax.ShapeDtypeStruct((B, 8, 128), jnp.int32),
            jax.ShapeDtypeStruct((B, 8, 128), jnp.float32),
        ],
    )(logits3)
    return perm, skip_gid, skip_w


def _gather_call(xflat, tok_pidx, skip_pidx, TOK_TOTAL, SKIP_TOTAL, PER_W,
                 SKIP_PER_W):
    NC, NS = 2, 16
    NW = NC * NS
    CK = 32
    n_chunk = PER_W // CK
    n_schunk = SKIP_PER_W // CK
    tok_last_base = TOK_TOTAL - PER_W

    mesh = plsc.VectorSubcoreMesh(core_axis_name="c", subcore_axis_name="s")

    @functools.partial(
        pl.kernel,
        mesh=mesh,
        out_type=[
            jax.ShapeDtypeStruct((6, TOK_TOTAL, 128), jnp.float32),
            jax.ShapeDtypeStruct((6, SKIP_TOTAL, 128), jnp.float32),
        ],
        scratch_types=[
            pltpu.VMEM((6 * PER_W,), jnp.int32),
            pltpu.VMEM((6 * SKIP_PER_W,), jnp.int32),
            pltpu.VMEM((6 * CK, 128), jnp.float32),
            pltpu.SemaphoreType.DMA,
            pltpu.SemaphoreType.DMA,
        ],
    )
    def gather_kernel(x_hbm, tokp_hbm, skipp_hbm, tok_out, skip_out,
                      tidx_v, sidx_v, rows_v, gsem, wsem):
        wid = lax.axis_index("s") * NC + lax.axis_index("c")
        base = jnp.minimum(wid * PER_W, tok_last_base)
        base = pl.multiple_of(base, 8)
        for k in range(6):
            pltpu.sync_copy(tokp_hbm.at[pl.ds(k * TOK_TOTAL + base, PER_W)],
                            tidx_v.at[pl.ds(k * PER_W, PER_W)])
        sbase = pl.multiple_of(wid * SKIP_PER_W, 8)
        for k in range(6):
            pltpu.sync_copy(
                skipp_hbm.at[pl.ds(k * SKIP_TOTAL + sbase, SKIP_PER_W)],
                sidx_v.at[pl.ds(k * SKIP_PER_W, SKIP_PER_W)])

        def tok_chunk(c, _):
            off = pl.multiple_of(c * CK, 8)
            cps = [pltpu.async_copy(
                x_hbm.at[tidx_v.at[pl.ds(k * PER_W + off, CK)]],
                rows_v.at[pl.ds(k * CK, CK)], gsem) for k in range(6)]
            for cp in cps:
                cp.wait()
            wps = [pltpu.async_copy(
                rows_v.at[pl.ds(k * CK, CK)],
                tok_out.at[k, pl.ds(base + off, CK)], wsem) for k in range(6)]
            for wp in wps:
                wp.wait()
            return _

        lax.fori_loop(0, n_chunk, tok_chunk, None)

        def skip_chunk(c, _):
            off = pl.multiple_of(c * CK, 8)
            cps = [pltpu.async_copy(
                x_hbm.at[sidx_v.at[pl.ds(k * SKIP_PER_W + off, CK)]],
                rows_v.at[pl.ds(k * CK, CK)], gsem) for k in range(6)]
            for cp in cps:
                cp.wait()
            wps = [pltpu.async_copy(
                rows_v.at[pl.ds(k * CK, CK)],
                skip_out.at[k, pl.ds(sbase + off, CK)], wsem) for k in range(6)]
            for wp in wps:
                wp.wait()
            return _

        lax.fori_loop(0, n_schunk, skip_chunk, None)

    return gather_kernel(xflat, tok_pidx, skip_pidx)


def _retile_body(src_ref, out_ref):
    pieces = [src_ref[k] for k in range(6)]
    out_ref[...] = jnp.concatenate(pieces, axis=-1)


def _retile_call(src, N, CH, D):
    return pl.pallas_call(
        _retile_body,
        grid=(N // CH,),
        in_specs=[pl.BlockSpec((6, CH, 128), lambda i: (0, i, 0))],
        out_specs=pl.BlockSpec((CH, D), lambda i: (i, 0)),
        out_shape=jax.ShapeDtypeStruct((N, D), jnp.float32),
    )(src)


def _summary_body(skip_ref, w_ref, out_ref):
    S = skip_ref[0].reshape(8, 128, skip_ref.shape[-1])
    wv = w_ref[0]
    acc = jnp.sum(S * wv[:, :, None], axis=(0, 1))
    out_ref[0, 0, :] = acc


def _summary_call(skip_pad3, skip_w, B, D):
    return pl.pallas_call(
        _summary_body,
        grid=(B,),
        in_specs=[
            pl.BlockSpec((1, 1024, D), lambda b: (b, 0, 0)),
            pl.BlockSpec((1, 8, 128), lambda b: (b, 0, 0)),
        ],
        out_specs=pl.BlockSpec((1, 1, D), lambda b: (b, 0, 0)),
        out_shape=jax.ShapeDtypeStruct((B, 1, D), jnp.float32),
    )(skip_pad3, skip_w)


def kernel(x, W, b):
    B, T, D = x.shape
    density = int(T * 0.9)
    k_skip = T - density

    logits3 = ((x @ W.T + b)[..., 0]).reshape(B, T // 128, 128)

    xlin = _linearize_call(x, B, T, D)

    perm, skip_gid, skip_w = _sort_call(logits3, B, T, k_skip)

    row_off = (jnp.arange(B, dtype=jnp.int32) * T)[:, None]
    perm_flat = perm.reshape(B, T) + row_off
    skip_gid_flat = (skip_gid.reshape(B, 1024) + row_off).reshape(-1)

    TOK_TOTAL = B * density
    SKIP_TOTAL = B * 1024
    NW = 32
    PER_W = 928
    SKIP_PER_W = SKIP_TOTAL // NW

    tok_gid = perm_flat[:, :density].reshape(-1)

    koff = (jnp.arange(6, dtype=jnp.int32) * (B * T))[:, None]
    tok_pidx = (tok_gid[None, :] + koff).reshape(-1)
    skip_pidx = (skip_gid_flat[None, :] + koff).reshape(-1)

    xflat = xlin.reshape(6 * B * T, 128)
    tok_lin, skip_lin = _gather_call(
        xflat, tok_pidx, skip_pidx, TOK_TOTAL, SKIP_TOTAL, PER_W, SKIP_PER_W)

    tok2d = _retile_call(tok_lin, TOK_TOTAL, 1552, D)
    skip2d = _retile_call(skip_lin, SKIP_TOTAL, 1024, D)

    tokens = tok2d.reshape(B, density, D)
    skip_pad3 = skip2d.reshape(B, 1024, D)
    skip_tokens = skip_pad3[:, :k_skip, :]

    summary = _summary_call(skip_pad3, skip_w, B, D)

    return (tokens, skip_tokens, summary)


# speedup vs baseline: 1.7498x; 1.7498x over previous
import functools

import jax
import jax.numpy as jnp
from jax import lax
from jax.experimental import pallas as pl
from jax.experimental.pallas import tpu as pltpu
from jax.experimental.pallas import tpu_sc as plsc


def _linearize_body(x_ref, out_ref):
    xb = x_ref[0]
    pieces = [xb[None, :, 128 * k:128 * (k + 1)] for k in range(6)]
    out_ref[...] = jnp.concatenate(pieces, axis=0)


def _linearize_call(x, B, T, D):
    TB = 2048
    n_blk = (B * T) // TB
    x3 = x.reshape(n_blk, TB, D)
    return pl.pallas_call(
        _linearize_body,
        grid=(n_blk,),
        in_specs=[pl.BlockSpec((1, TB, D), lambda i: (i, 0, 0))],
        out_specs=pl.BlockSpec((6, TB, 128), lambda i: (0, i, 0)),
        out_shape=jax.ShapeDtypeStruct((6, B * T, 128), jnp.float32),
    )(x3)


def _partner(arr, bit, axis):
    fwd = jnp.roll(arr, -bit, axis=axis)
    bwd = jnp.roll(arr, bit, axis=axis)
    io = lax.broadcasted_iota(jnp.int32, arr.shape, axis)
    take_fwd = (io & bit) == 0
    return jnp.where(take_fwd, fwd, bwd)


def _bitonic(keys, idxs, n, rows, lanes, descending):
    shape = keys.shape
    row_io = lax.broadcasted_iota(jnp.int32, shape, 1)
    lane_io = lax.broadcasted_iota(jnp.int32, shape, 2)
    flat_io = row_io * lanes + lane_io

    k = 2
    while k <= n:
        j = k // 2
        while j >= 1:
            if j < lanes:
                kp = _partner(keys, j, 2)
                ip = _partner(idxs, j, 2)
            else:
                rj = j // lanes
                kp = _partner(keys, rj, 1)
                ip = _partner(idxs, rj, 1)
            own_lower = (flat_io & j) == 0
            up = (flat_io & k) == 0
            if descending:
                own_first = (keys > kp) | ((keys == kp) & (idxs < ip))
            else:
                own_first = (keys < kp) | ((keys == kp) & (idxs < ip))
            keep_own = own_first == (own_lower == up)
            keys = jnp.where(keep_own, keys, kp)
            idxs = jnp.where(keep_own, idxs, ip)
            j //= 2
        k *= 2
    return keys, idxs


def _sort_body(T, K_SKIP, logits_ref, perm_ref, skipg_ref, skipw_ref):
    R = T // 128
    v = logits_ref[...]
    row_io = lax.broadcasted_iota(jnp.int32, v.shape, 1)
    lane_io = lax.broadcasted_iota(jnp.int32, v.shape, 2)
    idx = row_io * 128 + lane_io

    vs, isrt = _bitonic(v, idx, T, R, 128, descending=True)
    perm_ref[...] = isrt

    tv = vs[:, R - 8:, :]
    ti = isrt[:, R - 8:, :]
    tvs, tis = _bitonic(tv, ti, 1024, 8, 128, descending=False)
    skipg_ref[...] = tis

    fr = lax.broadcasted_iota(jnp.int32, tvs.shape, 1)
    fc = lax.broadcasted_iota(jnp.int32, tvs.shape, 2)
    fflat = fr * 128 + fc
    mask = fflat < K_SKIP
    mrow = (K_SKIP - 1) // 128
    mcol = (K_SKIP - 1) % 128
    m = tvs[:, mrow:mrow + 1, mcol:mcol + 1]
    e = jnp.exp(jnp.where(mask, tvs - m, -jnp.inf))
    s = jnp.sum(e, axis=(1, 2), keepdims=True)
    skipw_ref[...] = e / s


def _sort_call(logits3, B, T, K_SKIP):
    body = functools.partial(_sort_body, T, K_SKIP)
    R = T // 128
    perm, skip_gid, skip_w = pl.pallas_call(
        body,
        grid=(B,),
        in_specs=[pl.BlockSpec((1, R, 128), lambda b: (b, 0, 0))],
        out_specs=[
            pl.BlockSpec((1, R, 128), lambda b: (b, 0, 0)),
            pl.BlockSpec((1, 8, 128), lambda b: (b, 0, 0)),
            pl.BlockSpec((1, 8, 128), lambda b: (b, 0, 0)),
        ],
        out_shape=[
            jax.ShapeDtypeStruct((B, R, 128), jnp.int32),
            jax.ShapeDtypeStruct((B, 8, 128), jnp.int32),
            jax.ShapeDtypeStruct((B, 8, 128), jnp.float32),
        ],
    )(logits3)
    return perm, skip_gid, skip_w


def _gather_call(xflat, tok_pidx, skip_pidx, TOK_TOTAL, SKIP_TOTAL, PER_W,
                 SKIP_PER_W):
    NC, NS = 2, 16
    NW = NC * NS
    CK = 32
    n_chunk = PER_W // CK
    n_schunk = SKIP_PER_W // CK
    tok_last_base = TOK_TOTAL - PER_W

    mesh = plsc.VectorSubcoreMesh(core_axis_name="c", subcore_axis_name="s")

    @functools.partial(
        pl.kernel,
        mesh=mesh,
        out_type=[
            jax.ShapeDtypeStruct((6, TOK_TOTAL, 128), jnp.float32),
            jax.ShapeDtypeStruct((6, SKIP_TOTAL, 128), jnp.float32),
        ],
        scratch_types=[
            pltpu.VMEM((6 * PER_W,), jnp.int32),
            pltpu.VMEM((6 * SKIP_PER_W,), jnp.int32),
            pltpu.VMEM((6 * CK, 128), jnp.float32),
            pltpu.SemaphoreType.DMA,
            pltpu.SemaphoreType.DMA,
        ],
    )
    def gather_kernel(x_hbm, tokp_hbm, skipp_hbm, tok_out, skip_out,
                      tidx_v, sidx_v, rows_v, gsem, wsem):
        wid = lax.axis_index("s") * NC + lax.axis_index("c")
        base = jnp.minimum(wid * PER_W, tok_last_base)
        base = pl.multiple_of(base, 8)
        for k in range(6):
            pltpu.sync_copy(tokp_hbm.at[pl.ds(k * TOK_TOTAL + base, PER_W)],
                            tidx_v.at[pl.ds(k * PER_W, PER_W)])
        sbase = pl.multiple_of(wid * SKIP_PER_W, 8)
        for k in range(6):
            pltpu.sync_copy(
                skipp_hbm.at[pl.ds(k * SKIP_TOTAL + sbase, SKIP_PER_W)],
                sidx_v.at[pl.ds(k * SKIP_PER_W, SKIP_PER_W)])

        def tok_chunk(c, _):
            off = pl.multiple_of(c * CK, 8)
            cps = [pltpu.async_copy(
                x_hbm.at[tidx_v.at[pl.ds(k * PER_W + off, CK)]],
                rows_v.at[pl.ds(k * CK, CK)], gsem) for k in range(6)]
            for cp in cps:
                cp.wait()
            wps = [pltpu.async_copy(
                rows_v.at[pl.ds(k * CK, CK)],
                tok_out.at[k, pl.ds(base + off, CK)], wsem) for k in range(6)]
            for wp in wps:
                wp.wait()
            return _

        lax.fori_loop(0, n_chunk, tok_chunk, None)

        def skip_chunk(c, _):
            off = pl.multiple_of(c * CK, 8)
            cps = [pltpu.async_copy(
                x_hbm.at[sidx_v.at[pl.ds(k * SKIP_PER_W + off, CK)]],
                rows_v.at[pl.ds(k * CK, CK)], gsem) for k in range(6)]
            for cp in cps:
                cp.wait()
            wps = [pltpu.async_copy(
                rows_v.at[pl.ds(k * CK, CK)],
                skip_out.at[k, pl.ds(sbase + off, CK)], wsem) for k in range(6)]
            for wp in wps:
                wp.wait()
            return _

        lax.fori_loop(0, n_schunk, skip_chunk, None)

    return gather_kernel(xflat, tok_pidx, skip_pidx)


def _retile_body(src_ref, out_ref):
    pieces = [src_ref[k] for k in range(6)]
    out_ref[0] = jnp.concatenate(pieces, axis=-1)


def _retile_call(src, B, SEG, NOUT, CH, D):
    nblk = (NOUT + CH - 1) // CH
    seg_blk = SEG // CH
    return pl.pallas_call(
        _retile_body,
        grid=(B, nblk),
        in_specs=[pl.BlockSpec(
            (6, CH, 128), lambda b, i: (0, b * seg_blk + i, 0))],
        out_specs=pl.BlockSpec((1, CH, D), lambda b, i: (b, i, 0)),
        out_shape=jax.ShapeDtypeStruct((B, NOUT, D), jnp.float32),
    )(src)


def _summary_body(skip_ref, w_ref, out_ref):
    wv = w_ref[0]
    accs = []
    for k in range(6):
        s3 = skip_ref[k].reshape(8, 128, 128)
        accs.append(jnp.sum(s3 * wv[:, :, None], axis=(0, 1)))
    out_ref[0, 0, :] = jnp.concatenate(accs, axis=-1)


def _summary_call(skip_lin, skip_w, B, D):
    return pl.pallas_call(
        _summary_body,
        grid=(B,),
        in_specs=[
            pl.BlockSpec((6, 1024, 128), lambda b: (0, b, 0)),
            pl.BlockSpec((1, 8, 128), lambda b: (b, 0, 0)),
        ],
        out_specs=pl.BlockSpec((1, 1, D), lambda b: (b, 0, 0)),
        out_shape=jax.ShapeDtypeStruct((B, 1, D), jnp.float32),
    )(skip_lin, skip_w)


def kernel(x, W, b):
    B, T, D = x.shape
    density = int(T * 0.9)
    k_skip = T - density

    logits3 = ((x @ W.T + b)[..., 0]).reshape(B, T // 128, 128)

    xlin = _linearize_call(x, B, T, D)

    perm, skip_gid, skip_w = _sort_call(logits3, B, T, k_skip)

    row_off = (jnp.arange(B, dtype=jnp.int32) * T)[:, None]
    perm_flat = perm.reshape(B, T) + row_off
    skip_gid_flat = (skip_gid.reshape(B, 1024) + row_off).reshape(-1)

    NW = 32
    PER_W = 928
    density_pad = 8 * PER_W
    TOK_TOTAL = B * density_pad
    SKIP_TOTAL = B * 1024
    SKIP_PER_W = SKIP_TOTAL // NW

    tok_gid = perm_flat[:, :density_pad].reshape(-1)

    koff = (jnp.arange(6, dtype=jnp.int32) * (B * T))[:, None]
    tok_pidx = (tok_gid[None, :] + koff).reshape(-1)
    skip_pidx = (skip_gid_flat[None, :] + koff).reshape(-1)

    xflat = xlin.reshape(6 * B * T, 128)
    tok_lin, skip_lin = _gather_call(
        xflat, tok_pidx, skip_pidx, TOK_TOTAL, SKIP_TOTAL, PER_W, SKIP_PER_W)

    tokens = _retile_call(tok_lin, B, density_pad, density, 256, D)
    skip_tokens = _retile_call(skip_lin, B, 1024, k_skip, 128, D)

    summary = _summary_call(skip_lin, skip_w, B, D)

    return (tokens, skip_tokens, summary)


# speedup vs baseline: 1.9982x; 1.1419x over previous
import functools

import jax
import jax.numpy as jnp
from jax import lax
from jax.experimental import pallas as pl
from jax.experimental.pallas import tpu as pltpu
from jax.experimental.pallas import tpu_sc as plsc


def _linearize_body(x_ref, out_ref):
    xb = x_ref[0]
    pieces = [xb[None, :, 128 * k:128 * (k + 1)] for k in range(6)]
    out_ref[...] = jnp.concatenate(pieces, axis=0)


def _linearize_call(x, B, T, D):
    TB = 2048
    n_blk = (B * T) // TB
    x3 = x.reshape(n_blk, TB, D)
    return pl.pallas_call(
        _linearize_body,
        grid=(n_blk,),
        in_specs=[pl.BlockSpec((1, TB, D), lambda i: (i, 0, 0))],
        out_specs=pl.BlockSpec((6, TB, 128), lambda i: (0, i, 0)),
        out_shape=jax.ShapeDtypeStruct((6, B * T, 128), jnp.float32),
    )(x3)


def _partner(arr, bit, axis):
    fwd = jnp.roll(arr, -bit, axis=axis)
    bwd = jnp.roll(arr, bit, axis=axis)
    io = lax.broadcasted_iota(jnp.int32, arr.shape, axis)
    take_fwd = (io & bit) == 0
    return jnp.where(take_fwd, fwd, bwd)


def _bitonic(keys, idxs, n, rows, lanes, descending):
    shape = keys.shape
    row_io = lax.broadcasted_iota(jnp.int32, shape, 1)
    lane_io = lax.broadcasted_iota(jnp.int32, shape, 2)
    flat_io = row_io * lanes + lane_io

    k = 2
    while k <= n:
        j = k // 2
        while j >= 1:
            if j < lanes:
                kp = _partner(keys, j, 2)
                ip = _partner(idxs, j, 2)
            else:
                rj = j // lanes
                kp = _partner(keys, rj, 1)
                ip = _partner(idxs, rj, 1)
            own_lower = (flat_io & j) == 0
            up = (flat_io & k) == 0
            if descending:
                own_first = (keys > kp) | ((keys == kp) & (idxs < ip))
            else:
                own_first = (keys < kp) | ((keys == kp) & (idxs < ip))
            keep_own = own_first == (own_lower == up)
            keys = jnp.where(keep_own, keys, kp)
            idxs = jnp.where(keep_own, idxs, ip)
            j //= 2
        k *= 2
    return keys, idxs


def _sort_body(T, K_SKIP, logits_ref, perm_ref, skipg_ref, skipw_ref):
    R = T // 128
    v = logits_ref[...]
    row_io = lax.broadcasted_iota(jnp.int32, v.shape, 1)
    lane_io = lax.broadcasted_iota(jnp.int32, v.shape, 2)
    idx = row_io * 128 + lane_io

    vs, isrt = _bitonic(v, idx, T, R, 128, descending=True)
    perm_ref[...] = isrt

    tv = vs[:, R - 8:, :]
    ti = isrt[:, R - 8:, :]
    tvs, tis = _bitonic(tv, ti, 1024, 8, 128, descending=False)
    skipg_ref[...] = tis

    fr = lax.broadcasted_iota(jnp.int32, tvs.shape, 1)
    fc = lax.broadcasted_iota(jnp.int32, tvs.shape, 2)
    fflat = fr * 128 + fc
    mask = fflat < K_SKIP
    mrow = (K_SKIP - 1) // 128
    mcol = (K_SKIP - 1) % 128
    m = tvs[:, mrow:mrow + 1, mcol:mcol + 1]
    e = jnp.exp(jnp.where(mask, tvs - m, -jnp.inf))
    s = jnp.sum(e, axis=(1, 2), keepdims=True)
    skipw_ref[...] = e / s


def _sort_call(logits3, B, T, K_SKIP):
    body = functools.partial(_sort_body, T, K_SKIP)
    R = T // 128
    perm, skip_gid, skip_w = pl.pallas_call(
        body,
        grid=(B,),
        in_specs=[pl.BlockSpec((1, R, 128), lambda b: (b, 0, 0))],
        out_specs=[
            pl.BlockSpec((1, R, 128), lambda b: (b, 0, 0)),
            pl.BlockSpec((1, 8, 128), lambda b: (b, 0, 0)),
            pl.BlockSpec((1, 8, 128), lambda b: (b, 0, 0)),
        ],
        out_shape=[
            jax.ShapeDtypeStruct((B, R, 128), jnp.int32),
            jax.ShapeDtypeStruct((B, 8, 128), jnp.int32),
            jax.ShapeDtypeStruct((B, 8, 128), jnp.float32),
        ],
    )(logits3)
    return perm, skip_gid, skip_w


def _gather_call(x2d, tok_gid, skip_gid, TOK_TOTAL, SKIP_TOTAL, PER_W,
                 SKIP_PER_W):
    NC, NS = 2, 16
    NW = NC * NS
    CK = 32
    D = x2d.shape[-1]
    n_chunk = PER_W // CK
    n_schunk = SKIP_PER_W // CK

    mesh = plsc.VectorSubcoreMesh(core_axis_name="c", subcore_axis_name="s")

    @functools.partial(
        pl.kernel,
        mesh=mesh,
        out_type=[
            jax.ShapeDtypeStruct((6, TOK_TOTAL, 128), jnp.float32),
            jax.ShapeDtypeStruct((6, SKIP_TOTAL, 128), jnp.float32),
        ],
        scratch_types=[
            pltpu.VMEM((PER_W,), jnp.int32),
            pltpu.VMEM((SKIP_PER_W,), jnp.int32),
            pltpu.VMEM((CK, D), jnp.float32),
            pltpu.SemaphoreType.DMA,
            pltpu.SemaphoreType.DMA,
        ],
    )
    def gather_kernel(x_hbm, tokg_hbm, skipg_hbm, tok_out, skip_out,
                      tidx_v, sidx_v, rows_v, gsem, wsem):
        wid = lax.axis_index("s") * NC + lax.axis_index("c")
        base = pl.multiple_of(wid * PER_W, 8)
        pltpu.sync_copy(tokg_hbm.at[pl.ds(base, PER_W)], tidx_v)
        sbase = pl.multiple_of(wid * SKIP_PER_W, 8)
        pltpu.sync_copy(skipg_hbm.at[pl.ds(sbase, SKIP_PER_W)], sidx_v)

        def tok_chunk(c, _):
            off = pl.multiple_of(c * CK, 8)
            pltpu.async_copy(
                x_hbm.at[tidx_v.at[pl.ds(off, CK)]], rows_v, gsem).wait()
            wps = [pltpu.async_copy(
                rows_v.at[:, pl.ds(k * 128, 128)],
                tok_out.at[k, pl.ds(base + off, CK)], wsem) for k in range(6)]
            for wp in wps:
                wp.wait()
            return _

        lax.fori_loop(0, n_chunk, tok_chunk, None)

        def skip_chunk(c, _):
            off = pl.multiple_of(c * CK, 8)
            pltpu.async_copy(
                x_hbm.at[sidx_v.at[pl.ds(off, CK)]], rows_v, gsem).wait()
            wps = [pltpu.async_copy(
                rows_v.at[:, pl.ds(k * 128, 128)],
                skip_out.at[k, pl.ds(sbase + off, CK)], wsem) for k in range(6)]
            for wp in wps:
                wp.wait()
            return _

        lax.fori_loop(0, n_schunk, skip_chunk, None)

    return gather_kernel(x2d, tok_gid, skip_gid)


def _retile_body(src_ref, out_ref):
    pieces = [src_ref[k] for k in range(6)]
    out_ref[0] = jnp.concatenate(pieces, axis=-1)


def _retile_call(src, B, SEG, NOUT, CH, D):
    nblk = (NOUT + CH - 1) // CH
    seg_blk = SEG // CH
    return pl.pallas_call(
        _retile_body,
        grid=(B, nblk),
        in_specs=[pl.BlockSpec(
            (6, CH, 128), lambda b, i: (0, b * seg_blk + i, 0))],
        out_specs=pl.BlockSpec((1, CH, D), lambda b, i: (b, i, 0)),
        out_shape=jax.ShapeDtypeStruct((B, NOUT, D), jnp.float32),
    )(src)


def _summary_body(skip_ref, w_ref, out_ref):
    wv = w_ref[0]
    accs = []
    for k in range(6):
        s3 = skip_ref[k].reshape(8, 128, 128)
        accs.append(jnp.sum(s3 * wv[:, :, None], axis=(0, 1)))
    out_ref[0, 0, :] = jnp.concatenate(accs, axis=-1)


def _summary_call(skip_lin, skip_w, B, D):
    return pl.pallas_call(
        _summary_body,
        grid=(B,),
        in_specs=[
            pl.BlockSpec((6, 1024, 128), lambda b: (0, b, 0)),
            pl.BlockSpec((1, 8, 128), lambda b: (b, 0, 0)),
        ],
        out_specs=pl.BlockSpec((1, 1, D), lambda b: (b, 0, 0)),
        out_shape=jax.ShapeDtypeStruct((B, 1, D), jnp.float32),
    )(skip_lin, skip_w)


def kernel(x, W, b):
    B, T, D = x.shape
    density = int(T * 0.9)
    k_skip = T - density

    logits3 = ((x @ W.T + b)[..., 0]).reshape(B, T // 128, 128)

    perm, skip_gid, skip_w = _sort_call(logits3, B, T, k_skip)

    row_off = (jnp.arange(B, dtype=jnp.int32) * T)[:, None]
    perm_flat = perm.reshape(B, T) + row_off
    skip_gid_flat = (skip_gid.reshape(B, 1024) + row_off).reshape(-1)

    NW = 32
    PER_W = 928
    density_pad = 8 * PER_W
    TOK_TOTAL = B * density_pad
    SKIP_TOTAL = B * 1024
    SKIP_PER_W = SKIP_TOTAL // NW

    tok_gid = perm_flat[:, :density_pad].reshape(-1)

    x2d = x.reshape(B * T, D)
    tok_lin, skip_lin = _gather_call(
        x2d, tok_gid, skip_gid_flat, TOK_TOTAL, SKIP_TOTAL, PER_W, SKIP_PER_W)

    tokens = _retile_call(tok_lin, B, density_pad, density, 256, D)
    skip_tokens = _retile_call(skip_lin, B, 1024, k_skip, 128, D)

    summary = _summary_call(skip_lin, skip_w, B, D)

    return (tokens, skip_tokens, summary)


# speedup vs baseline: 2.1385x; 1.0702x over previous
import functools

import jax
import jax.numpy as jnp
from jax import lax
from jax.experimental import pallas as pl
from jax.experimental.pallas import tpu as pltpu
from jax.experimental.pallas import tpu_sc as plsc


def _linearize_body(x_ref, out_ref):
    xb = x_ref[0]
    pieces = [xb[None, :, 128 * k:128 * (k + 1)] for k in range(6)]
    out_ref[...] = jnp.concatenate(pieces, axis=0)


def _linearize_call(x, B, T, D):
    TB = 2048
    n_blk = (B * T) // TB
    x3 = x.reshape(n_blk, TB, D)
    return pl.pallas_call(
        _linearize_body,
        grid=(n_blk,),
        in_specs=[pl.BlockSpec((1, TB, D), lambda i: (i, 0, 0))],
        out_specs=pl.BlockSpec((6, TB, 128), lambda i: (0, i, 0)),
        out_shape=jax.ShapeDtypeStruct((6, B * T, 128), jnp.float32),
    )(x3)


def _partner(arr, bit, axis):
    fwd = jnp.roll(arr, -bit, axis=axis)
    bwd = jnp.roll(arr, bit, axis=axis)
    io = lax.broadcasted_iota(jnp.int32, arr.shape, axis)
    take_fwd = (io & bit) == 0
    return jnp.where(take_fwd, fwd, bwd)


def _bitonic(keys, idxs, n, rows, lanes, descending):
    shape = keys.shape
    row_io = lax.broadcasted_iota(jnp.int32, shape, 1)
    lane_io = lax.broadcasted_iota(jnp.int32, shape, 2)
    flat_io = row_io * lanes + lane_io

    k = 2
    while k <= n:
        j = k // 2
        while j >= 1:
            if j < lanes:
                kp = _partner(keys, j, 2)
                ip = _partner(idxs, j, 2)
            else:
                rj = j // lanes
                kp = _partner(keys, rj, 1)
                ip = _partner(idxs, rj, 1)
            own_lower = (flat_io & j) == 0
            up = (flat_io & k) == 0
            if descending:
                own_first = (keys > kp) | ((keys == kp) & (idxs < ip))
            else:
                own_first = (keys < kp) | ((keys == kp) & (idxs < ip))
            keep_own = own_first == (own_lower == up)
            keys = jnp.where(keep_own, keys, kp)
            idxs = jnp.where(keep_own, idxs, ip)
            j //= 2
        k *= 2
    return keys, idxs


def _sort_body(T, K_SKIP, logits_ref, perm_ref, skipg_ref, skipw_ref):
    R = T // 128
    v = logits_ref[...]
    row_io = lax.broadcasted_iota(jnp.int32, v.shape, 1)
    lane_io = lax.broadcasted_iota(jnp.int32, v.shape, 2)
    idx = row_io * 128 + lane_io

    vs, isrt = _bitonic(v, idx, T, R, 128, descending=True)
    perm_ref[...] = isrt

    tv = vs[:, R - 8:, :]
    ti = isrt[:, R - 8:, :]
    tvs, tis = _bitonic(tv, ti, 1024, 8, 128, descending=False)
    skipg_ref[...] = tis

    fr = lax.broadcasted_iota(jnp.int32, tvs.shape, 1)
    fc = lax.broadcasted_iota(jnp.int32, tvs.shape, 2)
    fflat = fr * 128 + fc
    mask = fflat < K_SKIP
    mrow = (K_SKIP - 1) // 128
    mcol = (K_SKIP - 1) % 128
    m = tvs[:, mrow:mrow + 1, mcol:mcol + 1]
    e = jnp.exp(jnp.where(mask, tvs - m, -jnp.inf))
    s = jnp.sum(e, axis=(1, 2), keepdims=True)
    skipw_ref[...] = e / s


def _sort_call(logits3, B, T, K_SKIP):
    body = functools.partial(_sort_body, T, K_SKIP)
    R = T // 128
    perm, skip_gid, skip_w = pl.pallas_call(
        body,
        in_specs=[pl.BlockSpec((B, R, 128), lambda: (0, 0, 0))],
        out_specs=[
            pl.BlockSpec((B, R, 128), lambda: (0, 0, 0)),
            pl.BlockSpec((B, 8, 128), lambda: (0, 0, 0)),
            pl.BlockSpec((B, 8, 128), lambda: (0, 0, 0)),
        ],
        out_shape=[
            jax.ShapeDtypeStruct((B, R, 128), jnp.int32),
            jax.ShapeDtypeStruct((B, 8, 128), jnp.int32),
            jax.ShapeDtypeStruct((B, 8, 128), jnp.float32),
        ],
    )(logits3)
    return perm, skip_gid, skip_w


def _gather_call(x2d, tok_gid, skip_gid, TOK_TOTAL, SKIP_TOTAL, PER_W,
                 SKIP_PER_W):
    NC, NS = 2, 16
    NW = NC * NS
    CK = 32
    D = x2d.shape[-1]
    n_chunk = PER_W // CK
    n_schunk = SKIP_PER_W // CK

    mesh = plsc.VectorSubcoreMesh(core_axis_name="c", subcore_axis_name="s")

    @functools.partial(
        pl.kernel,
        mesh=mesh,
        out_type=[
            jax.ShapeDtypeStruct((6, TOK_TOTAL, 128), jnp.float32),
            jax.ShapeDtypeStruct((6, SKIP_TOTAL, 128), jnp.float32),
        ],
        scratch_types=[
            pltpu.VMEM((PER_W,), jnp.int32),
            pltpu.VMEM((SKIP_PER_W,), jnp.int32),
            pltpu.VMEM((CK, D), jnp.float32),
            pltpu.SemaphoreType.DMA,
            pltpu.SemaphoreType.DMA,
        ],
    )
    def gather_kernel(x_hbm, tokg_hbm, skipg_hbm, tok_out, skip_out,
                      tidx_v, sidx_v, rows_v, gsem, wsem):
        wid = lax.axis_index("s") * NC + lax.axis_index("c")
        base = pl.multiple_of(wid * PER_W, 8)
        pltpu.sync_copy(tokg_hbm.at[pl.ds(base, PER_W)], tidx_v)
        sbase = pl.multiple_of(wid * SKIP_PER_W, 8)
        pltpu.sync_copy(skipg_hbm.at[pl.ds(sbase, SKIP_PER_W)], sidx_v)

        def tok_chunk(c, _):
            off = pl.multiple_of(c * CK, 8)
            pltpu.async_copy(
                x_hbm.at[tidx_v.at[pl.ds(off, CK)]], rows_v, gsem).wait()
            wps = [pltpu.async_copy(
                rows_v.at[:, pl.ds(k * 128, 128)],
                tok_out.at[k, pl.ds(base + off, CK)], wsem) for k in range(6)]
            for wp in wps:
                wp.wait()
            return _

        lax.fori_loop(0, n_chunk, tok_chunk, None)

        def skip_chunk(c, _):
            off = pl.multiple_of(c * CK, 8)
            pltpu.async_copy(
                x_hbm.at[sidx_v.at[pl.ds(off, CK)]], rows_v, gsem).wait()
            wps = [pltpu.async_copy(
                rows_v.at[:, pl.ds(k * 128, 128)],
                skip_out.at[k, pl.ds(sbase + off, CK)], wsem) for k in range(6)]
            for wp in wps:
                wp.wait()
            return _

        lax.fori_loop(0, n_schunk, skip_chunk, None)

    return gather_kernel(x2d, tok_gid, skip_gid)


def _retile_body(src_ref, out_ref):
    pieces = [src_ref[k] for k in range(6)]
    out_ref[0] = jnp.concatenate(pieces, axis=-1)


def _retile_call(src, B, SEG, NOUT, CH, D):
    nblk = (NOUT + CH - 1) // CH
    seg_blk = SEG // CH
    return pl.pallas_call(
        _retile_body,
        grid=(B, nblk),
        in_specs=[pl.BlockSpec(
            (6, CH, 128), lambda b, i: (0, b * seg_blk + i, 0))],
        out_specs=pl.BlockSpec((1, CH, D), lambda b, i: (b, i, 0)),
        out_shape=jax.ShapeDtypeStruct((B, NOUT, D), jnp.float32),
    )(src)


def _summary_body(skip_ref, w_ref, out_ref):
    wv = w_ref[0]
    accs = []
    for k in range(6):
        s3 = skip_ref[k].reshape(8, 128, 128)
        accs.append(jnp.sum(s3 * wv[:, :, None], axis=(0, 1)))
    out_ref[0, 0, :] = jnp.concatenate(accs, axis=-1)


def _summary_call(skip_lin, skip_w, B, D):
    return pl.pallas_call(
        _summary_body,
        grid=(B,),
        in_specs=[
            pl.BlockSpec((6, 1024, 128), lambda b: (0, b, 0)),
            pl.BlockSpec((1, 8, 128), lambda b: (b, 0, 0)),
        ],
        out_specs=pl.BlockSpec((1, 1, D), lambda b: (b, 0, 0)),
        out_shape=jax.ShapeDtypeStruct((B, 1, D), jnp.float32),
    )(skip_lin, skip_w)


def kernel(x, W, b):
    B, T, D = x.shape
    density = int(T * 0.9)
    k_skip = T - density

    logits3 = ((x @ W.T + b)[..., 0]).reshape(B, T // 128, 128)

    perm, skip_gid, skip_w = _sort_call(logits3, B, T, k_skip)

    row_off = (jnp.arange(B, dtype=jnp.int32) * T)[:, None]
    perm_flat = perm.reshape(B, T) + row_off
    skip_gid_flat = (skip_gid.reshape(B, 1024) + row_off).reshape(-1)

    NW = 32
    PER_W = 928
    density_pad = 8 * PER_W
    TOK_TOTAL = B * density_pad
    SKIP_TOTAL = B * 1024
    SKIP_PER_W = SKIP_TOTAL // NW

    tok_gid = perm_flat[:, :density_pad].reshape(-1)

    x2d = x.reshape(B * T, D)
    tok_lin, skip_lin = _gather_call(
        x2d, tok_gid, skip_gid_flat, TOK_TOTAL, SKIP_TOTAL, PER_W, SKIP_PER_W)

    tokens = _retile_call(tok_lin, B, density_pad, density, 256, D)
    skip_tokens = _retile_call(skip_lin, B, 1024, k_skip, 128, D)

    summary = _summary_call(skip_lin, skip_w, B, D)

    return (tokens, skip_tokens, summary)


# speedup vs baseline: 2.1862x; 1.0223x over previous
import functools

import jax
import jax.numpy as jnp
from jax import lax
from jax.experimental import pallas as pl
from jax.experimental.pallas import tpu as pltpu
from jax.experimental.pallas import tpu_sc as plsc


def _linearize_body(x_ref, out_ref):
    xb = x_ref[0]
    pieces = [xb[None, :, 128 * k:128 * (k + 1)] for k in range(6)]
    out_ref[...] = jnp.concatenate(pieces, axis=0)


def _linearize_call(x, B, T, D):
    TB = 2048
    n_blk = (B * T) // TB
    x3 = x.reshape(n_blk, TB, D)
    return pl.pallas_call(
        _linearize_body,
        grid=(n_blk,),
        in_specs=[pl.BlockSpec((1, TB, D), lambda i: (i, 0, 0))],
        out_specs=pl.BlockSpec((6, TB, 128), lambda i: (0, i, 0)),
        out_shape=jax.ShapeDtypeStruct((6, B * T, 128), jnp.float32),
    )(x3)


def _partner(arr, bit, axis):
    fwd = jnp.roll(arr, -bit, axis=axis)
    bwd = jnp.roll(arr, bit, axis=axis)
    io = lax.broadcasted_iota(jnp.int32, arr.shape, axis)
    take_fwd = (io & bit) == 0
    return jnp.where(take_fwd, fwd, bwd)


def _bitonic(keys, idxs, n, rows, lanes, descending):
    shape = keys.shape
    row_io = lax.broadcasted_iota(jnp.int32, shape, 1)
    lane_io = lax.broadcasted_iota(jnp.int32, shape, 2)
    flat_io = row_io * lanes + lane_io

    k = 2
    while k <= n:
        j = k // 2
        while j >= 1:
            if j < lanes:
                kp = _partner(keys, j, 2)
                ip = _partner(idxs, j, 2)
            else:
                rj = j // lanes
                kp = _partner(keys, rj, 1)
                ip = _partner(idxs, rj, 1)
            own_lower = (flat_io & j) == 0
            up = (flat_io & k) == 0
            if descending:
                own_first = (keys > kp) | ((keys == kp) & (idxs < ip))
            else:
                own_first = (keys < kp) | ((keys == kp) & (idxs < ip))
            keep_own = own_first == (own_lower == up)
            keys = jnp.where(keep_own, keys, kp)
            idxs = jnp.where(keep_own, idxs, ip)
            j //= 2
        k *= 2
    return keys, idxs


def _sort_body(T, K_SKIP, logits_ref, perm_ref, skipg_ref, skipw_ref):
    R = T // 128
    v = logits_ref[...]
    row_io = lax.broadcasted_iota(jnp.int32, v.shape, 1)
    lane_io = lax.broadcasted_iota(jnp.int32, v.shape, 2)
    idx = row_io * 128 + lane_io

    vs, isrt = _bitonic(v, idx, T, R, 128, descending=True)
    perm_ref[...] = isrt

    tv = vs[:, R - 8:, :]
    ti = isrt[:, R - 8:, :]
    tvs, tis = _bitonic(tv, ti, 1024, 8, 128, descending=False)
    skipg_ref[...] = tis

    fr = lax.broadcasted_iota(jnp.int32, tvs.shape, 1)
    fc = lax.broadcasted_iota(jnp.int32, tvs.shape, 2)
    fflat = fr * 128 + fc
    mask = fflat < K_SKIP
    mrow = (K_SKIP - 1) // 128
    mcol = (K_SKIP - 1) % 128
    m = tvs[:, mrow:mrow + 1, mcol:mcol + 1]
    e = jnp.exp(jnp.where(mask, tvs - m, -jnp.inf))
    s = jnp.sum(e, axis=(1, 2), keepdims=True)
    skipw_ref[...] = e / s


def _sort_call(logits3, B, T, K_SKIP):
    body = functools.partial(_sort_body, T, K_SKIP)
    R = T // 128
    perm, skip_gid, skip_w = pl.pallas_call(
        body,
        in_specs=[pl.BlockSpec((B, R, 128), lambda: (0, 0, 0))],
        out_specs=[
            pl.BlockSpec((B, R, 128), lambda: (0, 0, 0)),
            pl.BlockSpec((B, 8, 128), lambda: (0, 0, 0)),
            pl.BlockSpec((B, 8, 128), lambda: (0, 0, 0)),
        ],
        out_shape=[
            jax.ShapeDtypeStruct((B, R, 128), jnp.int32),
            jax.ShapeDtypeStruct((B, 8, 128), jnp.int32),
            jax.ShapeDtypeStruct((B, 8, 128), jnp.float32),
        ],
    )(logits3)
    return perm, skip_gid, skip_w


def _gather_call(x2d, tok_gid, skip_gid, TOK_TOTAL, SKIP_TOTAL, PER_W,
                 SKIP_PER_W):
    NC, NS = 2, 16
    NW = NC * NS
    CK = 64
    D = x2d.shape[-1]
    n_chunk = PER_W // CK
    n_schunk = SKIP_PER_W // CK

    mesh = plsc.VectorSubcoreMesh(core_axis_name="c", subcore_axis_name="s")

    @functools.partial(
        pl.kernel,
        mesh=mesh,
        out_type=[
            jax.ShapeDtypeStruct((6, TOK_TOTAL, 128), jnp.float32),
            jax.ShapeDtypeStruct((6, SKIP_TOTAL, 128), jnp.float32),
        ],
        scratch_types=[
            pltpu.VMEM((PER_W,), jnp.int32),
            pltpu.VMEM((SKIP_PER_W,), jnp.int32),
            pltpu.VMEM((CK, D), jnp.float32),
            pltpu.SemaphoreType.DMA,
            pltpu.SemaphoreType.DMA,
        ],
    )
    def gather_kernel(x_hbm, tokg_hbm, skipg_hbm, tok_out, skip_out,
                      tidx_v, sidx_v, rows_v, gsem, wsem):
        wid = lax.axis_index("s") * NC + lax.axis_index("c")
        base = pl.multiple_of(wid * PER_W, 8)
        pltpu.sync_copy(tokg_hbm.at[pl.ds(base, PER_W)], tidx_v)
        sbase = pl.multiple_of(wid * SKIP_PER_W, 8)
        pltpu.sync_copy(skipg_hbm.at[pl.ds(sbase, SKIP_PER_W)], sidx_v)

        def tok_chunk(c, _):
            off = pl.multiple_of(c * CK, 8)
            pltpu.async_copy(
                x_hbm.at[tidx_v.at[pl.ds(off, CK)]], rows_v, gsem).wait()
            wps = [pltpu.async_copy(
                rows_v.at[:, pl.ds(k * 128, 128)],
                tok_out.at[k, pl.ds(base + off, CK)], wsem) for k in range(6)]
            for wp in wps:
                wp.wait()
            return _

        lax.fori_loop(0, n_chunk, tok_chunk, None)

        def skip_chunk(c, _):
            off = pl.multiple_of(c * CK, 8)
            pltpu.async_copy(
                x_hbm.at[sidx_v.at[pl.ds(off, CK)]], rows_v, gsem).wait()
            wps = [pltpu.async_copy(
                rows_v.at[:, pl.ds(k * 128, 128)],
                skip_out.at[k, pl.ds(sbase + off, CK)], wsem) for k in range(6)]
            for wp in wps:
                wp.wait()
            return _

        lax.fori_loop(0, n_schunk, skip_chunk, None)

    return gather_kernel(x2d, tok_gid, skip_gid)


def _retile_body(src_ref, out_ref):
    pieces = [src_ref[k] for k in range(6)]
    out_ref[0] = jnp.concatenate(pieces, axis=-1)


def _retile_call(src, B, SEG, NOUT, CH, D):
    nblk = (NOUT + CH - 1) // CH
    seg_blk = SEG // CH
    return pl.pallas_call(
        _retile_body,
        grid=(B, nblk),
        in_specs=[pl.BlockSpec(
            (6, CH, 128), lambda b, i: (0, b * seg_blk + i, 0))],
        out_specs=pl.BlockSpec((1, CH, D), lambda b, i: (b, i, 0)),
        out_shape=jax.ShapeDtypeStruct((B, NOUT, D), jnp.float32),
    )(src)


def _summary_body(skip_ref, w_ref, out_ref):
    wv = w_ref[0]
    accs = []
    for k in range(6):
        s3 = skip_ref[k].reshape(8, 128, 128)
        accs.append(jnp.sum(s3 * wv[:, :, None], axis=(0, 1)))
    out_ref[0, 0, :] = jnp.concatenate(accs, axis=-1)


def _summary_call(skip_lin, skip_w, B, D):
    return pl.pallas_call(
        _summary_body,
        grid=(B,),
        in_specs=[
            pl.BlockSpec((6, 1024, 128), lambda b: (0, b, 0)),
            pl.BlockSpec((1, 8, 128), lambda b: (b, 0, 0)),
        ],
        out_specs=pl.BlockSpec((1, 1, D), lambda b: (b, 0, 0)),
        out_shape=jax.ShapeDtypeStruct((B, 1, D), jnp.float32),
    )(skip_lin, skip_w)


def kernel(x, W, b):
    B, T, D = x.shape
    density = int(T * 0.9)
    k_skip = T - density

    logits3 = ((x @ W.T + b)[..., 0]).reshape(B, T // 128, 128)

    perm, skip_gid, skip_w = _sort_call(logits3, B, T, k_skip)

    row_off = (jnp.arange(B, dtype=jnp.int32) * T)[:, None]
    perm_flat = perm.reshape(B, T) + row_off
    skip_gid_flat = (skip_gid.reshape(B, 1024) + row_off).reshape(-1)

    NW = 32
    PER_W = 960
    density_pad = 8 * PER_W
    TOK_TOTAL = B * density_pad
    SKIP_TOTAL = B * 1024
    SKIP_PER_W = SKIP_TOTAL // NW

    tok_gid = perm_flat[:, :density_pad].reshape(-1)

    x2d = x.reshape(B * T, D)
    tok_lin, skip_lin = _gather_call(
        x2d, tok_gid, skip_gid_flat, TOK_TOTAL, SKIP_TOTAL, PER_W, SKIP_PER_W)

    tokens = _retile_call(tok_lin, B, density_pad, density, 256, D)
    skip_tokens = _retile_call(skip_lin, B, 1024, k_skip, 128, D)

    summary = _summary_call(skip_lin, skip_w, B, D)

    return (tokens, skip_tokens, summary)
